# baseline (speedup 1.0000x reference)
import functools

import jax
import jax.numpy as jnp
from jax import lax
from jax.experimental import pallas as pl
from jax.experimental.pallas import tpu as pltpu
from jax.experimental.pallas import tpu_sc as plsc


def _pick_chunk(per_w: int) -> int:
    for c in range(min(128, per_w), 0, -1):
        if per_w % c == 0 and c % 8 == 0:
            return c
    raise ValueError(per_w)


@functools.cache
def _make_gather(n_rows: int, vocab: int, emb: int):
    info = plsc.get_sparse_core_info()
    nc, ns = info.num_cores, info.num_subcores
    nw = nc * ns
    assert n_rows % (nw * 8) == 0
    per_w = n_rows // nw
    chunk = _pick_chunk(per_w)
    n_chunks = per_w // chunk

    mesh = plsc.VectorSubcoreMesh(core_axis_name="c", subcore_axis_name="s")

    @functools.partial(
        pl.kernel,
        mesh=mesh,
        out_type=jax.ShapeDtypeStruct((n_rows, emb), jnp.float32),
        scratch_types=[
            pltpu.VMEM((n_chunks, chunk), jnp.int32),
            pltpu.VMEM((n_chunks, chunk, emb), jnp.float32),
            pltpu.SemaphoreType.DMA,
            pltpu.SemaphoreType.DMA,
            pltpu.SemaphoreType.DMA,
        ],
    )
    def gather_k(idx_hbm, table_hbm, out_hbm, idx_v, rows_v, isem, gsem, ssem):
        wid = lax.axis_index("s") * nc + lax.axis_index("c")
        base = wid * per_w
        for j in range(n_chunks):
            pltpu.async_copy(
                idx_hbm.at[pl.ds(base + j * chunk, chunk)], idx_v.at[j], isem
            )
        for j in range(n_chunks):
            pltpu.make_async_copy(
                idx_hbm.at[pl.ds(base + j * chunk, chunk)], idx_v.at[j], isem
            ).wait()
        for j in range(n_chunks):
            pltpu.async_copy(table_hbm.at[idx_v.at[j]], rows_v.at[j], gsem)
        for j in range(n_chunks):
            pltpu.make_async_copy(table_hbm.at[idx_v.at[j]], rows_v.at[j], gsem).wait()
            pltpu.async_copy(
                rows_v.at[j], out_hbm.at[pl.ds(base + j * chunk, chunk)], ssem
            )
        for j in range(n_chunks):
            pltpu.make_async_copy(
                rows_v.at[j], out_hbm.at[pl.ds(base + j * chunk, chunk)], ssem
            ).wait()

    return gather_k


def _mlp_body(
    g_ref, w1_ref, b1_ref, w2_ref, b2_ref, w3_ref, b3_ref, o_ref,
    w1s, w2s, w3s,
):
    @pl.when(pl.program_id(0) == 0)
    def _cast_weights():
        w1s[...] = w1_ref[...].astype(jnp.bfloat16)
        w2s[...] = w2_ref[...].astype(jnp.bfloat16)
        w3s[...] = w3_ref[...].astype(jnp.bfloat16)

    g = g_ref[...].astype(jnp.bfloat16)
    h = jnp.dot(g, w1s[...], preferred_element_type=jnp.float32)
    h = jnp.maximum(h + b1_ref[...], 0.0).astype(jnp.bfloat16)
    h = jnp.dot(h, w2s[...], preferred_element_type=jnp.float32)
    h = jnp.maximum(h + b2_ref[...], 0.0).astype(jnp.bfloat16)
    o = jnp.dot(h, w3s[...], preferred_element_type=jnp.float32)
    o_ref[...] = o + b3_ref[...]


@functools.cache
def _make_mlp(b: int, din: int, hid: int, dout: int, bm: int):
    grid = (b // bm,)
    return pl.pallas_call(
        _mlp_body,
        grid=grid,
        in_specs=[
            pl.BlockSpec((bm, din), lambda i: (i, 0)),
            pl.BlockSpec((din, hid), lambda i: (0, 0)),
            pl.BlockSpec((1, hid), lambda i: (0, 0)),
            pl.BlockSpec((hid, hid), lambda i: (0, 0)),
            pl.BlockSpec((1, hid), lambda i: (0, 0)),
            pl.BlockSpec((hid, dout), lambda i: (0, 0)),
            pl.BlockSpec((1, dout), lambda i: (0, 0)),
        ],
        out_specs=pl.BlockSpec((bm, dout), lambda i: (i, 0)),
        out_shape=jax.ShapeDtypeStruct((b, dout), jnp.float32),
        scratch_shapes=[
            pltpu.VMEM((din, hid), jnp.bfloat16),
            pltpu.VMEM((hid, hid), jnp.bfloat16),
            pltpu.VMEM((hid, dout), jnp.bfloat16),
        ],
        compiler_params=pltpu.CompilerParams(
            dimension_semantics=("arbitrary",),
        ),
    )


def kernel(x, E, W1, b1, W2, b2, W3, b3):
    b, win = x.shape
    vocab, emb = E.shape
    din, hid = W1.shape
    dout = W3.shape[1]

    xf = x.reshape(-1).astype(jnp.int32)
    g = _make_gather(b * win, vocab, emb)(xf, E).reshape(b, win * emb)

    mlp = _make_mlp(b, din, hid, dout, bm=512)
    return mlp(
        g, W1, b1.reshape(1, hid), W2, b2.reshape(1, hid), W3, b3.reshape(1, dout)
    )


# speedup vs baseline: 1.0077x; 1.0077x over previous
import functools

import jax
import jax.numpy as jnp
from jax import lax
from jax.experimental import pallas as pl
from jax.experimental.pallas import tpu as pltpu
from jax.experimental.pallas import tpu_sc as plsc


def _pick_chunk(per_w: int) -> int:
    for c in range(min(128, per_w), 0, -1):
        if per_w % c == 0 and c % 8 == 0:
            return c
    raise ValueError(per_w)


@functools.cache
def _make_gather(n_rows: int, vocab: int, emb: int):
    info = plsc.get_sparse_core_info()
    nc, ns = info.num_cores, info.num_subcores
    nw = nc * ns
    assert n_rows % (nw * 8) == 0
    per_w = n_rows // nw
    chunk = _pick_chunk(per_w)
    n_chunks = per_w // chunk

    mesh = plsc.VectorSubcoreMesh(core_axis_name="c", subcore_axis_name="s")

    @functools.partial(
        pl.kernel,
        mesh=mesh,
        out_type=jax.ShapeDtypeStruct((n_rows, emb), jnp.float32),
        scratch_types=[
            pltpu.VMEM((n_chunks, chunk), jnp.int32),
            pltpu.VMEM((n_chunks, chunk, emb), jnp.float32),
            pltpu.SemaphoreType.DMA,
            pltpu.SemaphoreType.DMA,
            pltpu.SemaphoreType.DMA,
        ],
    )
    def gather_k(idx_hbm, table_hbm, out_hbm, idx_v, rows_v, isem, gsem, ssem):
        wid = lax.axis_index("s") * nc + lax.axis_index("c")
        base = wid * per_w
        for j in range(n_chunks):
            pltpu.async_copy(
                idx_hbm.at[pl.ds(base + j * chunk, chunk)], idx_v.at[j], isem
            )
        for j in range(n_chunks):
            pltpu.make_async_copy(
                idx_hbm.at[pl.ds(base + j * chunk, chunk)], idx_v.at[j], isem
            ).wait()
        for j in range(n_chunks):
            pltpu.async_copy(table_hbm.at[idx_v.at[j]], rows_v.at[j], gsem)
        for j in range(n_chunks):
            pltpu.make_async_copy(table_hbm.at[idx_v.at[j]], rows_v.at[j], gsem).wait()
            pltpu.async_copy(
                rows_v.at[j], out_hbm.at[pl.ds(base + j * chunk, chunk)], ssem
            )
        for j in range(n_chunks):
            pltpu.make_async_copy(
                rows_v.at[j], out_hbm.at[pl.ds(base + j * chunk, chunk)], ssem
            ).wait()

    return gather_k


def _mlp_body(
    g_ref, w1_ref, b1_ref, w2_ref, b2_ref, w3_ref, b3_ref, o_ref,
    w1s, w2s, w3s,
):
    win, _, emb = g_ref.shape

    @pl.when(pl.program_id(0) == 0)
    def _cast_weights():
        w1s[...] = w1_ref[...].astype(jnp.bfloat16)
        w2s[...] = w2_ref[...].astype(jnp.bfloat16)
        w3s[...] = w3_ref[...].astype(jnp.bfloat16)

    h = jnp.dot(
        g_ref[0].astype(jnp.bfloat16),
        w1s[pl.ds(0, emb), :],
        preferred_element_type=jnp.float32,
    )
    for w in range(1, win):
        h += jnp.dot(
            g_ref[w].astype(jnp.bfloat16),
            w1s[pl.ds(w * emb, emb), :],
            preferred_element_type=jnp.float32,
        )
    h = jnp.maximum(h + b1_ref[...], 0.0).astype(jnp.bfloat16)
    h = jnp.dot(h, w2s[...], preferred_element_type=jnp.float32)
    h = jnp.maximum(h + b2_ref[...], 0.0).astype(jnp.bfloat16)
    o = jnp.dot(h, w3s[...], preferred_element_type=jnp.float32)
    o_ref[...] = o + b3_ref[...]


@functools.cache
def _make_mlp(b: int, din: int, hid: int, dout: int, bm: int):
    grid = (b // bm,)
    return pl.pallas_call(
        _mlp_body,
        grid=grid,
        in_specs=[
            pl.BlockSpec((din // 128, bm, 128), lambda i: (0, i, 0)),
            pl.BlockSpec((din, hid), lambda i: (0, 0)),
            pl.BlockSpec((1, hid), lambda i: (0, 0)),
            pl.BlockSpec((hid, hid), lambda i: (0, 0)),
            pl.BlockSpec((1, hid), lambda i: (0, 0)),
            pl.BlockSpec((hid, dout), lambda i: (0, 0)),
            pl.BlockSpec((1, dout), lambda i: (0, 0)),
        ],
        out_specs=pl.BlockSpec((bm, dout), lambda i: (i, 0)),
        out_shape=jax.ShapeDtypeStruct((b, dout), jnp.float32),
        scratch_shapes=[
            pltpu.VMEM((din, hid), jnp.bfloat16),
            pltpu.VMEM((hid, hid), jnp.bfloat16),
            pltpu.VMEM((hid, dout), jnp.bfloat16),
        ],
        compiler_params=pltpu.CompilerParams(
            dimension_semantics=("arbitrary",),
        ),
    )


def kernel(x, E, W1, b1, W2, b2, W3, b3):
    b, win = x.shape
    vocab, emb = E.shape
    din, hid = W1.shape
    dout = W3.shape[1]

    xt = jnp.transpose(x).reshape(-1).astype(jnp.int32)
    g = _make_gather(b * win, vocab, emb)(xt, E).reshape(win, b, emb)

    mlp = _make_mlp(b, din, hid, dout, bm=1024)
    return mlp(
        g, W1, b1.reshape(1, hid), W2, b2.reshape(1, hid), W3, b3.reshape(1, dout)
    )


# speedup vs baseline: 1.0937x; 1.0854x over previous
import functools

import jax
import jax.numpy as jnp
from jax import lax
from jax.experimental import pallas as pl
from jax.experimental.pallas import tpu as pltpu
from jax.experimental.pallas import tpu_sc as plsc


def _pick_chunk(per_w: int) -> int:
    for c in range(min(128, per_w), 0, -1):
        if per_w % c == 0 and c % 8 == 0:
            return c
    raise ValueError(per_w)


@functools.cache
def _make_gather(n_rows: int, vocab: int, emb: int):
    info = plsc.get_sparse_core_info()
    nc, ns = info.num_cores, info.num_subcores
    nw = nc * ns
    assert n_rows % (nw * 8) == 0
    per_w = n_rows // nw
    chunk = _pick_chunk(per_w)
    n_chunks = per_w // chunk

    mesh = plsc.VectorSubcoreMesh(core_axis_name="c", subcore_axis_name="s")

    @functools.partial(
        pl.kernel,
        mesh=mesh,
        out_type=jax.ShapeDtypeStruct((n_rows, emb), jnp.float32),
        scratch_types=[
            pltpu.VMEM((n_chunks, chunk), jnp.int32),
            pltpu.VMEM((n_chunks, chunk, emb), jnp.float32),
            pltpu.SemaphoreType.DMA,
            pltpu.SemaphoreType.DMA,
            pltpu.SemaphoreType.DMA,
        ],
    )
    def gather_k(idx_hbm, table_hbm, out_hbm, idx_v, rows_v, isem, gsem, ssem):
        wid = lax.axis_index("s") * nc + lax.axis_index("c")
        base = wid * per_w
        for j in range(n_chunks):
            pltpu.async_copy(
                idx_hbm.at[pl.ds(base + j * chunk, chunk)], idx_v.at[j], isem
            )
        for j in range(n_chunks):
            pltpu.make_async_copy(
                idx_hbm.at[pl.ds(base + j * chunk, chunk)], idx_v.at[j], isem
            ).wait()
        for j in range(n_chunks):
            pltpu.async_copy(table_hbm.at[idx_v.at[j]], rows_v.at[j], gsem)
        for j in range(n_chunks):
            pltpu.make_async_copy(table_hbm.at[idx_v.at[j]], rows_v.at[j], gsem).wait()
            pltpu.async_copy(
                rows_v.at[j], out_hbm.at[pl.ds(base + j * chunk, chunk)], ssem
            )
        for j in range(n_chunks):
            pltpu.make_async_copy(
                rows_v.at[j], out_hbm.at[pl.ds(base + j * chunk, chunk)], ssem
            ).wait()

    return gather_k


def _mlp_body(
    g_ref, w1_ref, b1_ref, w2_ref, b2_ref, w3_ref, b3_ref, o_ref,
    w1s, w2s, w3s,
):
    win, _, emb = g_ref.shape

    @pl.when(pl.program_id(0) == 0)
    def _cast_weights():
        w1s[...] = w1_ref[...].astype(jnp.bfloat16)
        w2s[...] = w2_ref[...].astype(jnp.bfloat16)
        w3s[...] = w3_ref[...].astype(jnp.bfloat16)

    h = None
    w = 0
    while w < win:
        k = min(2, win - w)
        a = g_ref[w].astype(jnp.bfloat16)
        if k == 2:
            a = jnp.concatenate([a, g_ref[w + 1].astype(jnp.bfloat16)], axis=1)
        part = jnp.dot(
            a, w1s[pl.ds(w * emb, k * emb), :], preferred_element_type=jnp.float32
        )
        h = part if h is None else h + part
        w += k
    h = jnp.maximum(h + b1_ref[...], 0.0).astype(jnp.bfloat16)
    h = jnp.dot(h, w2s[...], preferred_element_type=jnp.float32)
    h = jnp.maximum(h + b2_ref[...], 0.0).astype(jnp.bfloat16)
    o = jnp.dot(h, w3s[...], preferred_element_type=jnp.float32)
    o_ref[...] = o + b3_ref[...]


@functools.cache
def _make_mlp(b: int, din: int, hid: int, dout: int, bm: int):
    grid = (b // bm,)
    return pl.pallas_call(
        _mlp_body,
        grid=grid,
        in_specs=[
            pl.BlockSpec((din // 128, bm, 128), lambda i: (0, i, 0)),
            pl.BlockSpec((din, hid), lambda i: (0, 0)),
            pl.BlockSpec((1, hid), lambda i: (0, 0)),
            pl.BlockSpec((hid, hid), lambda i: (0, 0)),
            pl.BlockSpec((1, hid), lambda i: (0, 0)),
            pl.BlockSpec((hid, dout), lambda i: (0, 0)),
            pl.BlockSpec((1, dout), lambda i: (0, 0)),
        ],
        out_specs=pl.BlockSpec((bm, dout), lambda i: (i, 0)),
        out_shape=jax.ShapeDtypeStruct((b, dout), jnp.float32),
        scratch_shapes=[
            pltpu.VMEM((din, hid), jnp.bfloat16),
            pltpu.VMEM((hid, hid), jnp.bfloat16),
            pltpu.VMEM((hid, dout), jnp.bfloat16),
        ],
        compiler_params=pltpu.CompilerParams(
            dimension_semantics=("arbitrary",),
        ),
    )


def kernel(x, E, W1, b1, W2, b2, W3, b3):
    b, win = x.shape
    vocab, emb = E.shape
    din, hid = W1.shape
    dout = W3.shape[1]

    xt = jnp.transpose(x).reshape(-1).astype(jnp.int32)
    g = _make_gather(b * win, vocab, emb)(xt, E).reshape(win, b, emb)

    mlp = _make_mlp(b, din, hid, dout, bm=1024)
    return mlp(
        g, W1, b1.reshape(1, hid), W2, b2.reshape(1, hid), W3, b3.reshape(1, dout)
    )
